# baseline (speedup 1.0000x reference)
import functools

import jax
import jax.numpy as jnp
from jax import lax
from jax.experimental import pallas as pl
from jax.experimental.pallas import tpu as pltpu
from jax.experimental.pallas import tpu_sc as plsc

BLK = 128
NBUF = 4
AHEAD = 3
NTRS = 2


def _make_gather(V, D, N):
    info = plsc.get_sparse_core_info()
    NC, NS = info.num_cores, info.num_subcores
    NW = NC * NS
    assert N % (NW * BLK * NBUF) == 0
    b_per_w = N // NW
    n_blk = b_per_w // BLK
    n_groups = n_blk // NBUF
    n_dtile = D // 8
    n_tiles = (N * D) // (8 * 128)
    mesh = plsc.VectorSubcoreMesh(core_axis_name="c", subcore_axis_name="s")

    @functools.partial(
        pl.kernel,
        mesh=mesh,
        out_type=jax.ShapeDtypeStruct((n_tiles, 8, 128), jnp.float32),
        scratch_types=[
            pltpu.VMEM((b_per_w,), jnp.int32),
            pltpu.VMEM((NBUF, BLK, D), jnp.float32),
            pltpu.VMEM((NTRS, n_dtile, 8, 128), jnp.float32),
            [pltpu.SemaphoreType.DMA] * NBUF,
            [pltpu.SemaphoreType.DMA] * NTRS,
        ],
        compiler_params=pltpu.CompilerParams(
            use_tc_tiling_on_sc=False, needs_layout_passes=False
        ),
    )
    def gather_kernel(table_hbm, idx_hbm, out_hbm, idx_all, rows_v, trs_v,
                      sem_g, sem_o):
        wid = lax.axis_index("s") * NC + lax.axis_index("c")
        base = wid * b_per_w
        pltpu.sync_copy(idx_hbm.at[pl.ds(base, b_per_w)], idx_all)

        iota16 = lax.iota(jnp.int32, 16)
        row_vecs = [iota16 + (g * 16) for g in range(8)]

        def fire(c, s):
            pltpu.async_copy(
                table_hbm.at[idx_all.at[pl.ds(c * BLK, BLK)]],
                rows_v.at[s], sem_g[s],
            )

        def drain_gather(s):
            pltpu.make_async_copy(
                table_hbm.at[pl.ds(0, BLK)], rows_v.at[s], sem_g[s]
            ).wait()

        def transpose(s, t):
            rows = rows_v.at[s]
            trs = trs_v.at[t]

            def dtbody(dt, carry):
                for dr in range(8):
                    dcol = jnp.full((16,), dt * 8 + dr, jnp.int32)
                    for g in range(8):
                        v = plsc.load_gather(rows, [row_vecs[g], dcol])
                        trs[dt, dr, pl.ds(g * 16, 16)] = v
                return carry

            lax.fori_loop(0, n_dtile, dtbody, 0)

        def fire_out(c, t):
            m = wid * n_blk + c
            f = m >> 7
            bt = m & 127
            for dt in range(n_dtile):
                pltpu.async_copy(
                    trs_v.at[t].at[dt],
                    out_hbm.at[(f * n_dtile + dt) * 128 + bt],
                    sem_o[t],
                )

        def wait_out(t):
            pltpu.make_async_copy(
                trs_v.at[t], out_hbm.at[pl.ds(0, n_dtile)], sem_o[t]
            ).wait()

        for c0 in range(AHEAD):
            fire(c0, c0)

        def body(g, carry):
            for b in range(NBUF):
                c = g * NBUF + b
                t = b % NTRS

                @pl.when(c >= NTRS)
                def _():
                    wait_out(t)

                @pl.when(c + AHEAD < n_blk)
                def _():
                    fire(c + AHEAD, (b + AHEAD) % NBUF)

                drain_gather(b)
                transpose(b, t)
                fire_out(c, t)
            return carry

        lax.fori_loop(0, n_groups, body, 0)
        for t in range(NTRS):
            wait_out(t)

    return gather_kernel


def kernel(data, table):
    B, F = data.shape
    V, D = table.shape
    idx = data.T.reshape(-1).astype(jnp.int32)
    tiles = _make_gather(V, D, B * F)(table, idx)
    out5 = tiles.reshape(F, D // 8, B // 128, 8, 128)
    return out5.transpose(2, 4, 0, 1, 3).reshape(B, F, D)


# speedup vs baseline: 1.2315x; 1.2315x over previous
import functools

import jax
import jax.numpy as jnp
from jax import lax
from jax.experimental import pallas as pl
from jax.experimental.pallas import tpu as pltpu
from jax.experimental.pallas import tpu_sc as plsc

BLK = 128
NBUF = 4
AHEAD = 3
NTRS = 2


def _make_gather(V, D, N):
    info = plsc.get_sparse_core_info()
    NC, NS = info.num_cores, info.num_subcores
    NW = NC * NS
    assert N % (NW * BLK * NBUF) == 0
    b_per_w = N // NW
    n_blk = b_per_w // BLK
    n_groups = n_blk // NBUF
    n_dtile = D // 8
    n_tiles = (N * D) // (8 * 128)
    mesh = plsc.VectorSubcoreMesh(core_axis_name="c", subcore_axis_name="s")

    @functools.partial(
        pl.kernel,
        mesh=mesh,
        out_type=jax.ShapeDtypeStruct((n_tiles, 8, 128), jnp.float32),
        scratch_types=[
            pltpu.VMEM((b_per_w,), jnp.int32),
            pltpu.VMEM((NBUF, BLK, D), jnp.float32),
            pltpu.VMEM((NTRS, n_dtile, 8, 128), jnp.float32),
            [pltpu.SemaphoreType.DMA] * NBUF,
            [pltpu.SemaphoreType.DMA] * NTRS,
        ],
        compiler_params=pltpu.CompilerParams(
            use_tc_tiling_on_sc=False, needs_layout_passes=False
        ),
    )
    def gather_kernel(table_hbm, idx_hbm, out_hbm, idx_all, rows_v, trs_v,
                      sem_g, sem_o):
        wid = lax.axis_index("s") * NC + lax.axis_index("c")
        base = wid * b_per_w
        pltpu.sync_copy(idx_hbm.at[pl.ds(base, b_per_w)], idx_all)

        iota16 = lax.iota(jnp.int32, 16)
        row_vecs = [iota16 + (g * 16) for g in range(8)]

        def fire(c, s):
            pltpu.async_copy(
                table_hbm.at[idx_all.at[pl.ds(c * BLK, BLK)]],
                rows_v.at[s], sem_g[s],
            )

        def drain_gather(s):
            pltpu.make_async_copy(
                table_hbm.at[pl.ds(0, BLK)], rows_v.at[s], sem_g[s]
            ).wait()

        def transpose(s, t):
            rows = rows_v.at[s]
            trs = trs_v.at[t]

            @plsc.parallel_loop(0, D)
            def dbody(d):
                dcol = jnp.full((16,), d, jnp.int32)
                dt = d >> 3
                dr = d & 7
                for g in range(8):
                    v = plsc.load_gather(rows, [row_vecs[g], dcol])
                    trs[dt, dr, pl.ds(g * 16, 16)] = v

        def fire_out(c, t):
            m = wid * n_blk + c
            f = m >> 7
            bt = m & 127
            for dt in range(n_dtile):
                pltpu.async_copy(
                    trs_v.at[t].at[dt],
                    out_hbm.at[(f * n_dtile + dt) * 128 + bt],
                    sem_o[t],
                )

        def wait_out(t):
            pltpu.make_async_copy(
                trs_v.at[t], out_hbm.at[pl.ds(0, n_dtile)], sem_o[t]
            ).wait()

        for c0 in range(AHEAD):
            fire(c0, c0)

        def body(g, carry):
            for b in range(NBUF):
                c = g * NBUF + b
                t = b % NTRS

                @pl.when(c >= NTRS)
                def _():
                    wait_out(t)

                @pl.when(c + AHEAD < n_blk)
                def _():
                    fire(c + AHEAD, (b + AHEAD) % NBUF)

                drain_gather(b)
                transpose(b, t)
                fire_out(c, t)
            return carry

        lax.fori_loop(0, n_groups, body, 0)
        for t in range(NTRS):
            wait_out(t)

    return gather_kernel


def kernel(data, table):
    B, F = data.shape
    V, D = table.shape
    idx = data.T.reshape(-1).astype(jnp.int32)
    tiles = _make_gather(V, D, B * F)(table, idx)
    out5 = tiles.reshape(F, D // 8, B // 128, 8, 128)
    return out5.transpose(2, 4, 0, 1, 3).reshape(B, F, D)


# speedup vs baseline: 1.2327x; 1.0010x over previous
import functools

import jax
import jax.numpy as jnp
from jax import lax
from jax.experimental import pallas as pl
from jax.experimental.pallas import tpu as pltpu
from jax.experimental.pallas import tpu_sc as plsc

BLK = 128
NBUF = 4
AHEAD = 3
NTRS = 2


def _make_gather(V, D, N):
    info = plsc.get_sparse_core_info()
    NC, NS = info.num_cores, info.num_subcores
    NW = NC * NS
    assert N % (NW * BLK * NBUF) == 0
    b_per_w = N // NW
    n_blk = b_per_w // BLK
    n_groups = n_blk // NBUF
    n_dtile = D // 8
    n_tiles = (N * D) // (8 * 128)
    mesh = plsc.VectorSubcoreMesh(core_axis_name="c", subcore_axis_name="s")

    @functools.partial(
        pl.kernel,
        mesh=mesh,
        out_type=jax.ShapeDtypeStruct((n_tiles, 8, 128), jnp.float32),
        scratch_types=[
            pltpu.VMEM((b_per_w,), jnp.int32),
            pltpu.VMEM((NBUF, BLK, D), jnp.float32),
            pltpu.VMEM((NTRS, n_dtile, 8, 128), jnp.float32),
            [pltpu.SemaphoreType.DMA] * NBUF,
            [pltpu.SemaphoreType.DMA] * NTRS,
        ],
        compiler_params=pltpu.CompilerParams(
            use_tc_tiling_on_sc=False, needs_layout_passes=False
        ),
    )
    def gather_kernel(table_hbm, idx_hbm, out_hbm, idx_all, rows_v, trs_v,
                      sem_g, sem_o):
        wid = lax.axis_index("s") * NC + lax.axis_index("c")
        base = wid * b_per_w
        pltpu.sync_copy(idx_hbm.at[pl.ds(base, b_per_w)], idx_all)

        iota16 = lax.iota(jnp.int32, 16)
        row_vecs = [iota16 + (g * 16) for g in range(8)]

        def fire(c, s):
            pltpu.async_copy(
                table_hbm.at[idx_all.at[pl.ds(c * BLK, BLK)]],
                rows_v.at[s], sem_g[s],
            )

        def drain_gather(s):
            pltpu.make_async_copy(
                table_hbm.at[pl.ds(0, BLK)], rows_v.at[s], sem_g[s]
            ).wait()

        def transpose(s, t):
            rows = rows_v.at[s]
            trs = trs_v.at[t]

            @plsc.parallel_loop(0, D, unroll=4)
            def dbody(d):
                dcol = jnp.full((16,), d, jnp.int32)
                dt = d >> 3
                dr = d & 7
                for g in range(8):
                    v = plsc.load_gather(rows, [row_vecs[g], dcol])
                    trs[dt, dr, pl.ds(g * 16, 16)] = v

        def fire_out(c, t):
            m = wid * n_blk + c
            f = m >> 7
            bt = m & 127
            for dt in range(n_dtile):
                pltpu.async_copy(
                    trs_v.at[t].at[dt],
                    out_hbm.at[(f * n_dtile + dt) * 128 + bt],
                    sem_o[t],
                )

        def wait_out(t):
            pltpu.make_async_copy(
                trs_v.at[t], out_hbm.at[pl.ds(0, n_dtile)], sem_o[t]
            ).wait()

        for c0 in range(AHEAD):
            fire(c0, c0)

        def body(g, carry):
            for b in range(NBUF):
                c = g * NBUF + b
                t = b % NTRS

                @pl.when(c >= NTRS)
                def _():
                    wait_out(t)

                @pl.when(c + AHEAD < n_blk)
                def _():
                    fire(c + AHEAD, (b + AHEAD) % NBUF)

                drain_gather(b)
                transpose(b, t)
                fire_out(c, t)
            return carry

        lax.fori_loop(0, n_groups, body, 0)
        for t in range(NTRS):
            wait_out(t)

    return gather_kernel


def kernel(data, table):
    B, F = data.shape
    V, D = table.shape
    idx = data.T.reshape(-1).astype(jnp.int32)
    tiles = _make_gather(V, D, B * F)(table, idx)
    out5 = tiles.reshape(F, D // 8, B // 128, 8, 128)
    return out5.transpose(2, 4, 0, 1, 3).reshape(B, F, D)


# speedup vs baseline: 1.5251x; 1.2371x over previous
import functools

import jax
import jax.numpy as jnp
from jax import lax
from jax.experimental import pallas as pl
from jax.experimental.pallas import tpu as pltpu
from jax.experimental.pallas import tpu_sc as plsc

BLK = 128
NBUF = 4
AHEAD = 3
NTRS = 2


def _make_gather(V, D, N):
    info = plsc.get_sparse_core_info()
    NC, NS = info.num_cores, info.num_subcores
    NW = NC * NS
    assert N % (NW * BLK * NBUF) == 0
    b_per_w = N // NW
    n_blk = b_per_w // BLK
    n_groups = n_blk // NBUF
    n_dtile = D // 8
    n_tiles = (N * D) // (8 * 128)
    mesh = plsc.VectorSubcoreMesh(core_axis_name="c", subcore_axis_name="s")

    @functools.partial(
        pl.kernel,
        mesh=mesh,
        out_type=jax.ShapeDtypeStruct((n_tiles, 8, 128), jnp.float32),
        scratch_types=[
            pltpu.VMEM((b_per_w,), jnp.int32),
            pltpu.VMEM((NBUF, BLK, D), jnp.float32),
            pltpu.VMEM((NTRS, D, 128), jnp.float32),
            [pltpu.SemaphoreType.DMA] * NBUF,
            [pltpu.SemaphoreType.DMA] * NTRS,
        ],
        compiler_params=pltpu.CompilerParams(
            use_tc_tiling_on_sc=False, needs_layout_passes=False
        ),
    )
    def gather_kernel(table_hbm, idx_hbm, out_hbm, idx_all, rows_v, trs_v,
                      sem_g, sem_o):
        wid = lax.axis_index("s") * NC + lax.axis_index("c")
        base = wid * b_per_w
        pltpu.sync_copy(idx_hbm.at[pl.ds(base, b_per_w)], idx_all)

        iota16 = lax.iota(jnp.int32, 16)
        row_vecs = [iota16 + (g * 16) for g in range(8)]

        def fire(c, s):
            pltpu.async_copy(
                table_hbm.at[idx_all.at[pl.ds(c * BLK, BLK)]],
                rows_v.at[s], sem_g[s],
            )

        def drain_gather(s):
            pltpu.make_async_copy(
                table_hbm.at[pl.ds(0, BLK)], rows_v.at[s], sem_g[s]
            ).wait()

        def transpose(s, t):
            rows = rows_v.at[s]
            trs = trs_v.at[t]

            @plsc.parallel_loop(0, 16)
            def kbody(k):
                perm = (iota16 + k) & 15
                perms = [perm + (d0 * 16) for d0 in range(D // 16)]
                for rv in row_vecs:
                    for p in perms:
                        v = plsc.load_gather(rows, [rv, p])
                        plsc.store_scatter(trs, [p, rv], v)

        def fire_out(c, t):
            m = wid * n_blk + c
            f = m >> 7
            bt = m & 127
            for dt in range(n_dtile):
                pltpu.async_copy(
                    trs_v.at[t].at[pl.ds(dt * 8, 8)],
                    out_hbm.at[(f * n_dtile + dt) * 128 + bt],
                    sem_o[t],
                )

        def wait_out(t):
            for _ in range(n_dtile):
                pltpu.make_async_copy(
                    trs_v.at[t].at[pl.ds(0, 8)], out_hbm.at[0], sem_o[t]
                ).wait()

        for c0 in range(AHEAD):
            fire(c0, c0)

        def body(g, carry):
            for b in range(NBUF):
                c = g * NBUF + b
                t = b % NTRS

                @pl.when(c >= NTRS)
                def _():
                    wait_out(t)

                @pl.when(c + AHEAD < n_blk)
                def _():
                    fire(c + AHEAD, (b + AHEAD) % NBUF)

                drain_gather(b)
                transpose(b, t)
                fire_out(c, t)
            return carry

        lax.fori_loop(0, n_groups, body, 0)
        for t in range(NTRS):
            wait_out(t)

    return gather_kernel


def kernel(data, table):
    B, F = data.shape
    V, D = table.shape
    idx = data.T.reshape(-1).astype(jnp.int32)
    tiles = _make_gather(V, D, B * F)(table, idx)
    out5 = tiles.reshape(F, D // 8, B // 128, 8, 128)
    return out5.transpose(2, 4, 0, 1, 3).reshape(B, F, D)


# speedup vs baseline: 2.8104x; 1.8428x over previous
import functools

import jax
import jax.numpy as jnp
from jax import lax
from jax.experimental import pallas as pl
from jax.experimental.pallas import tpu as pltpu
from jax.experimental.pallas import tpu_sc as plsc

BLK = 128
NBUF = 4
AHEAD = 3
NTRS = 2


def _make_gather(V, D, N):
    info = plsc.get_sparse_core_info()
    NC, NS = info.num_cores, info.num_subcores
    NW = NC * NS
    assert N % (NW * BLK * NBUF) == 0
    b_per_w = N // NW
    n_blk = b_per_w // BLK
    n_groups = n_blk // NBUF
    n_dtile = D // 8
    n_tiles = (N * D) // (8 * 128)
    mesh = plsc.VectorSubcoreMesh(core_axis_name="c", subcore_axis_name="s")

    @functools.partial(
        pl.kernel,
        mesh=mesh,
        out_type=jax.ShapeDtypeStruct((n_tiles, 8, 128), jnp.float32),
        scratch_types=[
            pltpu.VMEM((b_per_w,), jnp.int32),
            pltpu.VMEM((NBUF, BLK, D), jnp.float32),
            pltpu.VMEM((NTRS, D, 128), jnp.float32),
            [pltpu.SemaphoreType.DMA] * NBUF,
            [pltpu.SemaphoreType.DMA] * NTRS,
        ],
        compiler_params=pltpu.CompilerParams(
            use_tc_tiling_on_sc=False, needs_layout_passes=False
        ),
    )
    def gather_kernel(table_hbm, idx_hbm, out_hbm, idx_all, rows_v, trs_v,
                      sem_g, sem_o):
        wid = lax.axis_index("s") * NC + lax.axis_index("c")
        base = wid * b_per_w
        pltpu.sync_copy(idx_hbm.at[pl.ds(base, b_per_w)], idx_all)

        iota16 = lax.iota(jnp.int32, 16)
        row_vecs = [iota16 + (g * 16) for g in range(8)]

        def fire(c, s):
            pltpu.async_copy(
                table_hbm.at[idx_all.at[pl.ds(c * BLK, BLK)]],
                rows_v.at[s], sem_g[s],
            )

        def drain_gather(s):
            pltpu.make_async_copy(
                table_hbm.at[pl.ds(0, BLK)], rows_v.at[s], sem_g[s]
            ).wait()

        def transpose(s, t):
            rows = rows_v.at[s]
            trs = trs_v.at[t]

            @plsc.parallel_loop(0, 16)
            def kbody(k):
                perm = (iota16 + k) & 15
                perms = [perm + (d0 * 16) for d0 in range(D // 16)]
                for rv in row_vecs:
                    for p in perms:
                        v = plsc.load_gather(rows, [rv, p])
                        plsc.store_scatter(trs, [p, rv], v)

        def fire_out(c, t):
            m = wid * n_blk + c
            f = m >> 7
            bt = m & 127
            for dt in range(n_dtile):
                pltpu.async_copy(
                    trs_v.at[t].at[pl.ds(dt * 8, 8)],
                    out_hbm.at[(f * n_dtile + dt) * 128 + bt],
                    sem_o[t],
                )

        def wait_out(t):
            for _ in range(n_dtile):
                pltpu.make_async_copy(
                    trs_v.at[t].at[pl.ds(0, 8)], out_hbm.at[0], sem_o[t]
                ).wait()

        for c0 in range(AHEAD):
            fire(c0, c0)

        def body(g, carry):
            for b in range(NBUF):
                c = g * NBUF + b
                t = b % NTRS

                @pl.when(c >= NTRS)
                def _():
                    wait_out(t)

                @pl.when(c + AHEAD < n_blk)
                def _():
                    fire(c + AHEAD, (b + AHEAD) % NBUF)

                drain_gather(b)
                transpose(b, t)
                fire_out(c, t)
            return carry

        lax.fori_loop(0, n_groups, body, 0)
        for t in range(NTRS):
            wait_out(t)

    return gather_kernel


def _make_linearize(V, D):
    info = plsc.get_sparse_core_info()
    NC, NS = info.num_cores, info.num_subcores
    NW = NC * NS
    n_full = V // 128
    tail = V - n_full * 128
    base_cols = n_full // NW
    extra = n_full - base_cols * NW
    n_groups = (base_cols + (1 if extra else 0) + 1) // 2
    mesh = plsc.VectorSubcoreMesh(core_axis_name="c", subcore_axis_name="s")

    @functools.partial(
        pl.kernel,
        mesh=mesh,
        out_type=jax.ShapeDtypeStruct((V * D,), jnp.float32),
        scratch_types=[
            [pltpu.VMEM((D, 128), jnp.float32)] * 2,
            [pltpu.VMEM((128 * D,), jnp.float32)] * 2,
            [pltpu.SemaphoreType.DMA] * 2,
            [pltpu.SemaphoreType.DMA] * 2,
        ],
        compiler_params=pltpu.CompilerParams(
            use_tc_tiling_on_sc=True, needs_layout_passes=False
        ),
    )
    def lin_kernel(tblT_hbm, out_hbm, src_v, trs_v, sem_i, sem_o):
        wid = lax.axis_index("s") * NC + lax.axis_index("c")
        lo = wid * base_cols + jnp.minimum(wid, extra)
        cnt = base_cols + (wid < extra).astype(jnp.int32)

        iota16 = lax.iota(jnp.int32, 16)
        row_vecs = [iota16 + (rb * 16) for rb in range(8)]
        rv32 = [(iota16 + (rb * 16)) * D for rb in range(8)]

        def fire_load(rt, b):
            pltpu.async_copy(
                tblT_hbm.at[:, pl.ds(rt * 128, 128)], src_v[b], sem_i[b]
            )

        def wait_load(b):
            pltpu.make_async_copy(
                tblT_hbm.at[:, pl.ds(0, 128)], src_v[b], sem_i[b]
            ).wait()

        def transpose(b, n_rb):
            src = src_v[b]
            trs = trs_v[b]

            @plsc.parallel_loop(0, 16)
            def kbody(k):
                perm = (iota16 + k) & 15
                for d0 in range(D // 16):
                    dvec = perm + (d0 * 16)
                    for rb in range(n_rb):
                        v = plsc.load_gather(src, [dvec, row_vecs[rb]])
                        plsc.store_scatter(trs, [rv32[rb] + dvec], v)

        def fire_store(rt, b):
            pltpu.async_copy(
                trs_v[b], out_hbm.at[pl.ds(rt * (128 * D), 128 * D)], sem_o[b]
            )

        def wait_store(b):
            pltpu.make_async_copy(
                trs_v[b], out_hbm.at[pl.ds(0, 128 * D)], sem_o[b]
            ).wait()

        fire_load(lo, 0)

        def body(g, carry):
            for b in range(2):
                i = g * 2 + b

                @pl.when(i < cnt)
                def _():
                    wait_load(b)

                    @pl.when(i + 1 < cnt)
                    def _():
                        fire_load(lo + i + 1, 1 - b)

                    @pl.when(i >= 2)
                    def _():
                        wait_store(b)

                    transpose(b, 8)
                    fire_store(lo + i, b)

            return carry

        lax.fori_loop(0, n_groups, body, 0)
        for b in range(2):
            wait_store(b)


    return lin_kernel


def kernel(data, table):
    B, F = data.shape
    V, D = table.shape
    idx = data.T.reshape(-1).astype(jnp.int32)
    tbl_lin = _make_linearize(V, D)(table.T)
    t0 = (V // 128) * 128
    if t0 < V:
        tbl_lin = lax.dynamic_update_slice(
            tbl_lin, table[t0:].reshape(-1), (t0 * D,)
        )
    tiles = _make_gather(V, D, B * F)(tbl_lin.reshape(V, D), idx)
    out5 = tiles.reshape(F, D // 8, B // 128, 8, 128)
    return out5.transpose(2, 4, 0, 1, 3).reshape(B, F, D)


# speedup vs baseline: 3.9177x; 1.3940x over previous
import functools

import jax
import jax.numpy as jnp
from jax import lax
from jax.experimental import pallas as pl
from jax.experimental.pallas import tpu as pltpu
from jax.experimental.pallas import tpu_sc as plsc

BLK = 128
NBUF = 4
AHEAD = 3
NTRS = 2


def _make_gather(V, D, N):
    info = plsc.get_sparse_core_info()
    NC, NS = info.num_cores, info.num_subcores
    NW = NC * NS
    assert N % (NW * BLK * NBUF) == 0
    b_per_w = N // NW
    n_blk = b_per_w // BLK
    n_groups = n_blk // NBUF
    n_dtile = D // 8
    n_tiles = (N * D) // (8 * 128)
    mesh = plsc.VectorSubcoreMesh(core_axis_name="c", subcore_axis_name="s")

    @functools.partial(
        pl.kernel,
        mesh=mesh,
        out_type=jax.ShapeDtypeStruct((n_tiles, 8, 128), jnp.float32),
        scratch_types=[
            pltpu.VMEM((b_per_w,), jnp.int32),
            pltpu.VMEM((NBUF, BLK, D), jnp.float32),
            pltpu.VMEM((NTRS, D, 128), jnp.float32),
            [pltpu.SemaphoreType.DMA] * NBUF,
            [pltpu.SemaphoreType.DMA] * NTRS,
        ],
        compiler_params=pltpu.CompilerParams(
            use_tc_tiling_on_sc=False, needs_layout_passes=False
        ),
    )
    def gather_kernel(table_hbm, idx_hbm, out_hbm, idx_all, rows_v, trs_v,
                      sem_g, sem_o):
        wid = lax.axis_index("s") * NC + lax.axis_index("c")
        base = wid * b_per_w
        pltpu.sync_copy(idx_hbm.at[pl.ds(base, b_per_w)], idx_all)

        iota16 = lax.iota(jnp.int32, 16)
        row_vecs = [iota16 + (g * 16) for g in range(8)]

        def fire(c, s):
            pltpu.async_copy(
                table_hbm.at[idx_all.at[pl.ds(c * BLK, BLK)]],
                rows_v.at[s], sem_g[s],
            )

        def drain_gather(s):
            pltpu.make_async_copy(
                table_hbm.at[pl.ds(0, BLK)], rows_v.at[s], sem_g[s]
            ).wait()

        def transpose(s, t):
            rows = rows_v.at[s]
            trs = trs_v.at[t]

            @plsc.parallel_loop(0, 16)
            def kbody(k):
                perm = (iota16 + k) & 15
                perms = [perm + (d0 * 16) for d0 in range(D // 16)]
                for rv in row_vecs:
                    for p in perms:
                        v = plsc.load_gather(rows, [rv, p])
                        plsc.store_scatter(trs, [p, rv], v)

        def fire_out(c, t):
            m = wid * n_blk + c
            f = m >> 7
            bt = m & 127
            for dt in range(n_dtile):
                pltpu.async_copy(
                    trs_v.at[t].at[pl.ds(dt * 8, 8)],
                    out_hbm.at[(f * n_dtile + dt) * 128 + bt],
                    sem_o[t],
                )

        def wait_out(t):
            for _ in range(n_dtile):
                pltpu.make_async_copy(
                    trs_v.at[t].at[pl.ds(0, 8)], out_hbm.at[0], sem_o[t]
                ).wait()

        for c0 in range(AHEAD):
            fire(c0, c0)

        def body(g, carry):
            for b in range(NBUF):
                c = g * NBUF + b
                t = b % NTRS

                @pl.when(c >= NTRS)
                def _():
                    wait_out(t)

                @pl.when(c + AHEAD < n_blk)
                def _():
                    fire(c + AHEAD, (b + AHEAD) % NBUF)

                drain_gather(b)
                transpose(b, t)
                fire_out(c, t)
            return carry

        lax.fori_loop(0, n_groups, body, 0)
        for t in range(NTRS):
            wait_out(t)

    return gather_kernel


def _make_linearize(V, D):
    info = plsc.get_sparse_core_info()
    NC, NS = info.num_cores, info.num_subcores
    NW = NC * NS
    n_full = V // 128
    tail = V - n_full * 128
    base_cols = n_full // NW
    extra = n_full - base_cols * NW
    n_groups = (base_cols + (1 if extra else 0) + 5) // 6
    mesh = plsc.VectorSubcoreMesh(core_axis_name="c", subcore_axis_name="s")

    @functools.partial(
        pl.kernel,
        mesh=mesh,
        out_type=jax.ShapeDtypeStruct((V * D,), jnp.float32),
        scratch_types=[
            [pltpu.VMEM((D, 128), jnp.float32)] * 3,
            [pltpu.VMEM((128 * D,), jnp.float32)] * 2,
            [pltpu.SemaphoreType.DMA] * 3,
            [pltpu.SemaphoreType.DMA] * 2,
        ],
        compiler_params=pltpu.CompilerParams(
            use_tc_tiling_on_sc=True, needs_layout_passes=False
        ),
    )
    def lin_kernel(tblT_hbm, out_hbm, src_v, trs_v, sem_i, sem_o):
        wid = lax.axis_index("s") * NC + lax.axis_index("c")
        lo = wid * base_cols + jnp.minimum(wid, extra)
        cnt = base_cols + (wid < extra).astype(jnp.int32)

        iota16 = lax.iota(jnp.int32, 16)
        row_vecs = [iota16 + (rb * 16) for rb in range(8)]
        rv32 = [(iota16 + (rb * 16)) * D for rb in range(8)]

        def fire_load(rt, b):
            pltpu.async_copy(
                tblT_hbm.at[:, pl.ds(rt * 128, 128)], src_v[b], sem_i[b]
            )

        def wait_load(b):
            pltpu.make_async_copy(
                tblT_hbm.at[:, pl.ds(0, 128)], src_v[b], sem_i[b]
            ).wait()

        def transpose(sb, tb, n_rb):
            src = src_v[sb]
            trs = trs_v[tb]

            @plsc.parallel_loop(0, 16, unroll=2)
            def kbody(k):
                perm = (iota16 + k) & 15
                for d0 in range(D // 16):
                    dvec = perm + (d0 * 16)
                    for rb in range(n_rb):
                        v = plsc.load_gather(src, [dvec, row_vecs[rb]])
                        plsc.store_scatter(trs, [rv32[rb] + dvec], v)

        def fire_store(rt, b):
            pltpu.async_copy(
                trs_v[b], out_hbm.at[pl.ds(rt * (128 * D), 128 * D)], sem_o[b]
            )

        def wait_store(b):
            pltpu.make_async_copy(
                trs_v[b], out_hbm.at[pl.ds(0, 128 * D)], sem_o[b]
            ).wait()

        fire_load(lo, 0)
        fire_load(lo + 1, 1)

        def body(g, carry):
            for b6 in range(6):
                i = g * 6 + b6
                sb = b6 % 3
                tb = b6 % 2

                @pl.when(i + 2 < cnt)
                def _():
                    fire_load(lo + i + 2, (b6 + 2) % 3)

                @pl.when(i < cnt)
                def _():
                    wait_load(sb)

                    @pl.when(i >= 2)
                    def _():
                        wait_store(tb)

                    transpose(sb, tb, 8)
                    fire_store(lo + i, tb)

            return carry

        lax.fori_loop(0, n_groups, body, 0)
        for b in range(2):
            wait_store(b)


    return lin_kernel


def kernel(data, table):
    B, F = data.shape
    V, D = table.shape
    idx = data.T.reshape(-1).astype(jnp.int32)
    tbl_lin = _make_linearize(V, D)(table.T)
    t0 = (V // 128) * 128
    if t0 < V:
        tbl_lin = lax.dynamic_update_slice(
            tbl_lin, table[t0:].reshape(-1), (t0 * D,)
        )
    tiles = _make_gather(V, D, B * F)(tbl_lin.reshape(V, D), idx)
    out5 = tiles.reshape(F, D // 8, B // 128, 8, 128)
    return out5.transpose(2, 4, 0, 1, 3).reshape(B, F, D)


# speedup vs baseline: 3.9185x; 1.0002x over previous
import functools

import jax
import jax.numpy as jnp
from jax import lax
from jax.experimental import pallas as pl
from jax.experimental.pallas import tpu as pltpu
from jax.experimental.pallas import tpu_sc as plsc

BLK = 128
NBUF = 4
AHEAD = 3
NTRS = 2


def _make_gather(V, D, N):
    info = plsc.get_sparse_core_info()
    NC, NS = info.num_cores, info.num_subcores
    NW = NC * NS
    assert N % (NW * BLK * NBUF) == 0
    b_per_w = N // NW
    n_blk = b_per_w // BLK
    n_groups = n_blk // NBUF
    n_dtile = D // 8
    n_tiles = (N * D) // (8 * 128)
    mesh = plsc.VectorSubcoreMesh(core_axis_name="c", subcore_axis_name="s")

    @functools.partial(
        pl.kernel,
        mesh=mesh,
        out_type=jax.ShapeDtypeStruct((n_tiles, 8, 128), jnp.float32),
        scratch_types=[
            pltpu.VMEM((b_per_w,), jnp.int32),
            pltpu.VMEM((NBUF, BLK, D), jnp.float32),
            pltpu.VMEM((NTRS, D, 128), jnp.float32),
            [pltpu.SemaphoreType.DMA] * NBUF,
            [pltpu.SemaphoreType.DMA] * NTRS,
        ],
        compiler_params=pltpu.CompilerParams(
            use_tc_tiling_on_sc=False, needs_layout_passes=False
        ),
    )
    def gather_kernel(table_hbm, idx_hbm, out_hbm, idx_all, rows_v, trs_v,
                      sem_g, sem_o):
        wid = lax.axis_index("s") * NC + lax.axis_index("c")
        base = wid * b_per_w
        pltpu.sync_copy(idx_hbm.at[pl.ds(base, b_per_w)], idx_all)

        iota16 = lax.iota(jnp.int32, 16)
        row_vecs = [iota16 + (g * 16) for g in range(8)]

        def fire(c, s):
            pltpu.async_copy(
                table_hbm.at[idx_all.at[pl.ds(c * BLK, BLK)]],
                rows_v.at[s], sem_g[s],
            )

        def drain_gather(s):
            pltpu.make_async_copy(
                table_hbm.at[pl.ds(0, BLK)], rows_v.at[s], sem_g[s]
            ).wait()

        def transpose(s, t):
            rows = rows_v.at[s]
            trs = trs_v.at[t]

            @plsc.parallel_loop(0, 16, unroll=2)
            def kbody(k):
                perm = (iota16 + k) & 15
                perms = [perm + (d0 * 16) for d0 in range(D // 16)]
                for rv in row_vecs:
                    for p in perms:
                        v = plsc.load_gather(rows, [rv, p])
                        plsc.store_scatter(trs, [p, rv], v)

        def fire_out(c, t):
            m = wid * n_blk + c
            f = m >> 7
            bt = m & 127
            for dt in range(n_dtile):
                pltpu.async_copy(
                    trs_v.at[t].at[pl.ds(dt * 8, 8)],
                    out_hbm.at[(f * n_dtile + dt) * 128 + bt],
                    sem_o[t],
                )

        def wait_out(t):
            for _ in range(n_dtile):
                pltpu.make_async_copy(
                    trs_v.at[t].at[pl.ds(0, 8)], out_hbm.at[0], sem_o[t]
                ).wait()

        for c0 in range(AHEAD):
            fire(c0, c0)

        def body(g, carry):
            for b in range(NBUF):
                c = g * NBUF + b
                t = b % NTRS

                @pl.when(c >= NTRS)
                def _():
                    wait_out(t)

                @pl.when(c + AHEAD < n_blk)
                def _():
                    fire(c + AHEAD, (b + AHEAD) % NBUF)

                drain_gather(b)
                transpose(b, t)
                fire_out(c, t)
            return carry

        lax.fori_loop(0, n_groups, body, 0)
        for t in range(NTRS):
            wait_out(t)

    return gather_kernel


def _make_linearize(V, D):
    info = plsc.get_sparse_core_info()
    NC, NS = info.num_cores, info.num_subcores
    NW = NC * NS
    n_full = V // 128
    tail = V - n_full * 128
    base_cols = n_full // NW
    extra = n_full - base_cols * NW
    n_groups = (base_cols + (1 if extra else 0) + 5) // 6
    mesh = plsc.VectorSubcoreMesh(core_axis_name="c", subcore_axis_name="s")

    @functools.partial(
        pl.kernel,
        mesh=mesh,
        out_type=jax.ShapeDtypeStruct((V * D,), jnp.float32),
        scratch_types=[
            [pltpu.VMEM((D, 128), jnp.float32)] * 3,
            [pltpu.VMEM((128 * D,), jnp.float32)] * 2,
            [pltpu.SemaphoreType.DMA] * 3,
            [pltpu.SemaphoreType.DMA] * 2,
        ],
        compiler_params=pltpu.CompilerParams(
            use_tc_tiling_on_sc=True, needs_layout_passes=False
        ),
    )
    def lin_kernel(tblT_hbm, out_hbm, src_v, trs_v, sem_i, sem_o):
        wid = lax.axis_index("s") * NC + lax.axis_index("c")
        lo = wid * base_cols + jnp.minimum(wid, extra)
        cnt = base_cols + (wid < extra).astype(jnp.int32)

        iota16 = lax.iota(jnp.int32, 16)
        row_vecs = [iota16 + (rb * 16) for rb in range(8)]
        rv32 = [(iota16 + (rb * 16)) * D for rb in range(8)]

        def fire_load(rt, b):
            pltpu.async_copy(
                tblT_hbm.at[:, pl.ds(rt * 128, 128)], src_v[b], sem_i[b]
            )

        def wait_load(b):
            pltpu.make_async_copy(
                tblT_hbm.at[:, pl.ds(0, 128)], src_v[b], sem_i[b]
            ).wait()

        def transpose(sb, tb, n_rb):
            src = src_v[sb]
            trs = trs_v[tb]

            @plsc.parallel_loop(0, 16, unroll=4)
            def kbody(k):
                perm = (iota16 + k) & 15
                for d0 in range(D // 16):
                    dvec = perm + (d0 * 16)
                    for rb in range(n_rb):
                        v = plsc.load_gather(src, [dvec, row_vecs[rb]])
                        plsc.store_scatter(trs, [rv32[rb] + dvec], v)

        def fire_store(rt, b):
            pltpu.async_copy(
                trs_v[b], out_hbm.at[pl.ds(rt * (128 * D), 128 * D)], sem_o[b]
            )

        def wait_store(b):
            pltpu.make_async_copy(
                trs_v[b], out_hbm.at[pl.ds(0, 128 * D)], sem_o[b]
            ).wait()

        fire_load(lo, 0)
        fire_load(lo + 1, 1)

        def body(g, carry):
            for b6 in range(6):
                i = g * 6 + b6
                sb = b6 % 3
                tb = b6 % 2

                @pl.when(i + 2 < cnt)
                def _():
                    fire_load(lo + i + 2, (b6 + 2) % 3)

                @pl.when(i < cnt)
                def _():
                    wait_load(sb)

                    @pl.when(i >= 2)
                    def _():
                        wait_store(tb)

                    transpose(sb, tb, 8)
                    fire_store(lo + i, tb)

            return carry

        lax.fori_loop(0, n_groups, body, 0)
        for b in range(2):
            wait_store(b)


    return lin_kernel


def kernel(data, table):
    B, F = data.shape
    V, D = table.shape
    idx = data.T.reshape(-1).astype(jnp.int32)
    tbl_lin = _make_linearize(V, D)(table.T)
    t0 = (V // 128) * 128
    if t0 < V:
        tbl_lin = lax.dynamic_update_slice(
            tbl_lin, table[t0:].reshape(-1), (t0 * D,)
        )
    tiles = _make_gather(V, D, B * F)(tbl_lin.reshape(V, D), idx)
    out5 = tiles.reshape(F, D // 8, B // 128, 8, 128)
    return out5.transpose(2, 4, 0, 1, 3).reshape(B, F, D)
